# baseline (speedup 1.0000x reference)
import jax
import jax.numpy as jnp
from jax import lax
from jax.experimental import pallas as pl
from jax.experimental.pallas import tpu as pltpu

N = 1000000
ROWS = 1000
COLS = 1000
BR = 40
NB = ROWS // BR
BIG = 2**30


def _body(x_ref, out_ref, max_ref, idx_ref):
    i = pl.program_id(0)

    @pl.when(i < NB)
    def _phase1():
        xv = x_ref[...]
        lm = jnp.max(xv)

        @pl.when((i == 0) | (lm > max_ref[0]))
        def _new_max():
            rows = lax.broadcasted_iota(jnp.int32, (BR, COLS), 0)
            cols = lax.broadcasted_iota(jnp.int32, (BR, COLS), 1)
            lin = (rows + i * BR) * COLS + cols
            cand = jnp.where(xv == lm, lin, BIG)
            max_ref[0] = lm
            idx_ref[0] = jnp.min(cand)

    @pl.when(i >= NB)
    def _phase2():
        j = i - NB
        out_ref[...] = jnp.zeros((BR, COLS), jnp.int32)
        idx = idx_ref[0]
        r0 = idx // COLS

        @pl.when((r0 >= j * BR) & (r0 < (j + 1) * BR))
        def _patch():
            c = idx - r0 * COLS
            cols = lax.broadcasted_iota(jnp.int32, (1, COLS), 1)
            out_ref[pl.ds(r0 - j * BR, 1), :] = (cols == c).astype(jnp.int32)


def kernel(x):
    x2 = x.reshape(ROWS, COLS)
    out = pl.pallas_call(
        _body,
        grid=(2 * NB,),
        in_specs=[pl.BlockSpec((BR, COLS), lambda i: (jnp.minimum(i, NB - 1), 0))],
        out_specs=pl.BlockSpec((BR, COLS), lambda i: (jnp.maximum(i - NB, 0), 0)),
        out_shape=jax.ShapeDtypeStruct((ROWS, COLS), jnp.int32),
        scratch_shapes=[
            pltpu.SMEM((1,), jnp.float32),
            pltpu.SMEM((1,), jnp.int32),
        ],
    )(x2)
    return out.reshape(N)


# speedup vs baseline: 1.7450x; 1.7450x over previous
import jax
import jax.numpy as jnp
from jax import lax
from jax.experimental import pallas as pl
from jax.experimental.pallas import tpu as pltpu

N = 1000000
ROWS = 1000
COLS = 1000
BR = 200
NB = ROWS // BR
BIG = 2**30


def _body(x_ref, out_ref, max_ref, idx_ref):
    i = pl.program_id(0)

    @pl.when(i < NB)
    def _phase1():
        xv = x_ref[...]
        lm = jnp.max(xv)

        @pl.when((i == 0) | (lm > max_ref[0]))
        def _new_max():
            rows = lax.broadcasted_iota(jnp.int32, (BR, COLS), 0)
            cols = lax.broadcasted_iota(jnp.int32, (BR, COLS), 1)
            lin = (rows + i * BR) * COLS + cols
            cand = jnp.where(xv == lm, lin, BIG)
            max_ref[0] = lm
            idx_ref[0] = jnp.min(cand)

    @pl.when(i >= NB)
    def _phase2():
        j = i - NB
        out_ref[...] = jnp.zeros((BR, COLS), jnp.int32)
        idx = idx_ref[0]
        r0 = idx // COLS

        @pl.when((r0 >= j * BR) & (r0 < (j + 1) * BR))
        def _patch():
            c = idx - r0 * COLS
            cols = lax.broadcasted_iota(jnp.int32, (1, COLS), 1)
            out_ref[pl.ds(r0 - j * BR, 1), :] = (cols == c).astype(jnp.int32)


def kernel(x):
    x2 = x.reshape(ROWS, COLS)
    out = pl.pallas_call(
        _body,
        grid=(2 * NB,),
        in_specs=[pl.BlockSpec((BR, COLS), lambda i: (jnp.minimum(i, NB - 1), 0))],
        out_specs=pl.BlockSpec((BR, COLS), lambda i: (jnp.maximum(i - NB, 0), 0)),
        out_shape=jax.ShapeDtypeStruct((ROWS, COLS), jnp.int32),
        scratch_shapes=[
            pltpu.SMEM((1,), jnp.float32),
            pltpu.SMEM((1,), jnp.int32),
        ],
    )(x2)
    return out.reshape(N)
